# Initial kernel scaffold; baseline (speedup 1.0000x reference)
import functools

import jax
import jax.numpy as jnp
from jax import lax
from jax.experimental import pallas as pl
from jax.experimental.pallas import tpu as pltpu
from jax.experimental.pallas import tpu_sc as plsc

_N = 10000
_E = 160000
_L = 3
_H = 4
_D = 64
_HD = _H * _D
_NC = 2
_NS = 16
_EPS = _E // _NS
_CH = 80
_NCHUNK = _EPS // _CH
_ROWW = 144
_NZ = _N // _NS
_ZB = 125


def _fw_body(f_ref, w_ref, b_ref, o_ref):
    o_ref[...] = (
        jnp.dot(f_ref[...], w_ref[...], preferred_element_type=jnp.float32)
        + b_ref[...]
    )


def _fw_linear(features, w, b):
    n = features.shape[0]
    bn = 1000
    return pl.pallas_call(
        _fw_body,
        grid=(n // bn,),
        in_specs=[
            pl.BlockSpec((bn, _D), lambda i: (i, 0)),
            pl.BlockSpec((_D, _HD), lambda i: (0, 0)),
            pl.BlockSpec((_HD,), lambda i: (0,)),
        ],
        out_specs=pl.BlockSpec((bn, _HD), lambda i: (i, 0)),
        out_shape=jax.ShapeDtypeStruct((n, _HD), jnp.float32),
    )(features, w, b)


def _fin_body(acc_ref, o_ref):
    for h in range(_H):
        c, j = divmod(h, 2)
        num = acc_ref[c, :, 64 * j:64 * (j + 1)]
        den = acc_ref[c, :, 128 + j:129 + j] + 1e-16
        o_ref[:, 64 * h:64 * (h + 1)] = num / den


def _finalize(acc):
    bn = 1000
    return pl.pallas_call(
        _fin_body,
        grid=(_N // bn,),
        in_specs=[pl.BlockSpec((_NC, bn, _ROWW), lambda i: (0, i, 0))],
        out_specs=pl.BlockSpec((bn, _HD), lambda i: (i, 0)),
        out_shape=jax.ShapeDtypeStruct((_N, _HD), jnp.float32),
    )(acc)


def _sc_body(fw_hbm, emi0_hbm, emi1_hbm, emi2_hbm, dst_hbm, attn_hbm,
             out_hbm,
             idx0, idx1, idx2, g0, g1, g2, row_buf, dst_buf, attn_v,
             zero_buf, acc, sem0, sem1, sem2):
    c = lax.axis_index("c")
    s = lax.axis_index("s")

    pltpu.sync_copy(attn_hbm.at[pl.ds(c * 128, 128)], attn_v)
    av = [attn_v[pl.ds(16 * k, 16)] for k in range(8)]
    lane = lax.iota(jnp.int32, 16)
    zvec = jnp.zeros((16,), jnp.float32)

    def _zrow(r, _):
        for k in range(_ROWW // 16):
            zero_buf[r, pl.ds(16 * k, 16)] = zvec
        return 0
    lax.fori_loop(0, _ZB, _zrow, 0)
    for k in range(_NZ // _ZB):
        pltpu.sync_copy(zero_buf, acc.at[pl.ds(s * _NZ + k * _ZB, _ZB)])
    plsc.subcore_barrier()

    base0 = s * _EPS

    def _chunk(i, _):
        base = base0 + i * _CH
        pltpu.sync_copy(emi0_hbm.at[pl.ds(base, _CH)], idx0)
        pltpu.sync_copy(emi1_hbm.at[pl.ds(base, _CH)], idx1)
        pltpu.sync_copy(emi2_hbm.at[pl.ds(base, _CH)], idx2)
        pltpu.sync_copy(dst_hbm.at[pl.ds(base, _CH)], dst_buf)
        for ib in (idx0, idx1, idx2):
            for k in range(_CH // 16):
                ib[pl.ds(16 * k, 16)] = ib[pl.ds(16 * k, 16)] * 2 + c
        cp0 = pltpu.async_copy(fw_hbm.at[idx0], g0, sem0)
        cp1 = pltpu.async_copy(fw_hbm.at[idx1], g1, sem1)
        cp2 = pltpu.async_copy(fw_hbm.at[idx2], g2, sem2)
        cp0.wait()
        cp1.wait()
        cp2.wait()

        def _edge(e, _):
            r = [
                jnp.maximum(
                    jnp.maximum(g0[e, pl.ds(16 * k, 16)],
                                g1[e, pl.ds(16 * k, 16)]),
                    g2[e, pl.ds(16 * k, 16)])
                for k in range(8)
            ]
            s0 = r[0] * av[0] + r[1] * av[1] + r[2] * av[2] + r[3] * av[3]
            s1 = r[4] * av[4] + r[5] * av[5] + r[6] * av[6] + r[7] * av[7]
            a0 = jnp.sum(s0)
            a1 = jnp.sum(s1)
            v0 = jnp.full((16,), a0)
            v1 = jnp.full((16,), a1)
            p0 = jnp.exp(jnp.where(v0 > 0, v0, v0 * 0.01))
            p1 = jnp.exp(jnp.where(v1 > 0, v1, v1 * 0.01))
            for k in range(4):
                row_buf[e, pl.ds(16 * k, 16)] = r[k] * p0
            for k in range(4, 8):
                row_buf[e, pl.ds(16 * k, 16)] = r[k] * p1
            tail = jnp.where(lane == 0, p0, jnp.where(lane == 1, p1, zvec))
            row_buf[e, pl.ds(128, 16)] = tail
            return 0

        lax.fori_loop(0, _CH, _edge, 0)
        pltpu.sync_copy(row_buf, acc.at[dst_buf], add=True)
        return 0

    lax.fori_loop(0, _NCHUNK, _chunk, 0)
    plsc.subcore_barrier()

    for k in range(_NZ // _ZB):
        pltpu.sync_copy(acc.at[pl.ds(s * _NZ + k * _ZB, _ZB)],
                        out_hbm.at[c, pl.ds(s * _NZ + k * _ZB, _ZB)])


_sc_kernel = functools.partial(
    pl.kernel,
    _sc_body,
    out_type=jax.ShapeDtypeStruct((_NC, _N, _ROWW), jnp.float32),
    mesh=plsc.VectorSubcoreMesh(core_axis_name="c", subcore_axis_name="s"),
    scratch_types=[
        pltpu.VMEM((_CH,), jnp.int32),
        pltpu.VMEM((_CH,), jnp.int32),
        pltpu.VMEM((_CH,), jnp.int32),
        pltpu.VMEM((_CH, 128), jnp.float32),
        pltpu.VMEM((_CH, 128), jnp.float32),
        pltpu.VMEM((_CH, 128), jnp.float32),
        pltpu.VMEM((_CH, _ROWW), jnp.float32),
        pltpu.VMEM((_CH,), jnp.int32),
        pltpu.VMEM((128,), jnp.float32),
        pltpu.VMEM((_ZB, _ROWW), jnp.float32),
        pltpu.VMEM_SHARED((_N, _ROWW), jnp.float32),
        pltpu.SemaphoreType.DMA,
        pltpu.SemaphoreType.DMA,
        pltpu.SemaphoreType.DMA,
    ],
)


def kernel(features, edge_index, type_mask, edge_metapath_indices,
           W_rnn, b_rnn, attn):
    del type_mask
    n, d = features.shape
    e, l = edge_metapath_indices.shape
    h = attn.shape[1]
    assert (n, e, l, h, d) == (_N, _E, _L, _H, _D)

    fw = _fw_linear(features, W_rnn, b_rnn)
    fw2 = fw.reshape(_N * _NC, _HD // _NC)
    emi0 = edge_metapath_indices[:, 0]
    emi1 = edge_metapath_indices[:, 1]
    emi2 = edge_metapath_indices[:, 2]
    dst = edge_index[1]
    attn_flat = attn.reshape(_HD)

    acc = _sc_kernel()(fw2, emi0, emi1, emi2, dst, attn_flat)
    out = _finalize(acc)
    return out.reshape(_N, _H, _D)


# speedup vs baseline: 13.6746x; 13.6746x over previous
import functools

import jax
import jax.numpy as jnp
from jax import lax
from jax.experimental import pallas as pl
from jax.experimental.pallas import tpu as pltpu
from jax.experimental.pallas import tpu_sc as plsc

_N = 10000
_E = 160000
_L = 3
_H = 4
_D = 64
_HD = _H * _D
_NC = 2
_NS = 16
_NPH = _H // _NC
_EPS = _E // _NS
_CH = 80
_NCHUNK = _EPS // _CH
_ROWW = 80
_NZ = _N // _NS
_ZB = 125


def _fw_body(f_ref, w_ref, b_ref, o_ref):
    o_ref[...] = (
        jnp.dot(f_ref[...], w_ref[...], preferred_element_type=jnp.float32)
        + b_ref[...]
    )


def _fw_linear(features, w, b):
    n = features.shape[0]
    bn = 1000
    return pl.pallas_call(
        _fw_body,
        grid=(n // bn,),
        in_specs=[
            pl.BlockSpec((bn, _D), lambda i: (i, 0)),
            pl.BlockSpec((_D, _HD), lambda i: (0, 0)),
            pl.BlockSpec((_HD,), lambda i: (0,)),
        ],
        out_specs=pl.BlockSpec((bn, _HD), lambda i: (i, 0)),
        out_shape=jax.ShapeDtypeStruct((n, _HD), jnp.float32),
    )(features, w, b)


def _fin_body(acc_ref, o_ref):
    for h in range(_H):
        num = acc_ref[h, :, :_D]
        den = acc_ref[h, :, _D:_D + 1] + 1e-16
        o_ref[:, _D * h:_D * (h + 1)] = num / den


def _finalize(acc):
    bn = 1000
    return pl.pallas_call(
        _fin_body,
        grid=(_N // bn,),
        in_specs=[pl.BlockSpec((_H, bn, _ROWW), lambda i: (0, i, 0))],
        out_specs=pl.BlockSpec((bn, _HD), lambda i: (i, 0)),
        out_shape=jax.ShapeDtypeStruct((_N, _HD), jnp.float32),
    )(acc)


def _sc_body(fw_hbm, emi0_hbm, emi1_hbm, emi2_hbm, dst_hbm, attn_hbm,
             out_hbm,
             idx0, idx1, idx2, g0, g1, g2, row_buf, dst_buf, attn_v,
             zero_buf, acc, sem0, sem1, sem2):
    c = lax.axis_index("c")
    s = lax.axis_index("s")

    pltpu.sync_copy(attn_hbm.at[pl.ds(c * 2 * _D, 2 * _D)], attn_v)
    lane = lax.iota(jnp.int32, 16)
    zvec = jnp.zeros((16,), jnp.float32)

    def _zrow(r, _):
        for k in range(_ROWW // 16):
            zero_buf[r, pl.ds(16 * k, 16)] = zvec
        return 0
    lax.fori_loop(0, _ZB, _zrow, 0)

    base0 = s * _EPS

    for q in range(_NPH):
        h = 2 * c + q
        av = [attn_v[pl.ds(_D * q + 16 * k, 16)] for k in range(_D // 16)]

        for k in range(_NZ // _ZB):
            pltpu.sync_copy(zero_buf, acc.at[pl.ds(s * _NZ + k * _ZB, _ZB)])
        plsc.subcore_barrier()

        def _chunk(i, _):
            base = base0 + i * _CH
            pltpu.sync_copy(emi0_hbm.at[pl.ds(base, _CH)], idx0)
            pltpu.sync_copy(emi1_hbm.at[pl.ds(base, _CH)], idx1)
            pltpu.sync_copy(emi2_hbm.at[pl.ds(base, _CH)], idx2)
            pltpu.sync_copy(dst_hbm.at[pl.ds(base, _CH)], dst_buf)
            for ib in (idx0, idx1, idx2):
                for k in range(_CH // 16):
                    ib[pl.ds(16 * k, 16)] = ib[pl.ds(16 * k, 16)] * 4 + h
            cp0 = pltpu.async_copy(fw_hbm.at[idx0], g0, sem0)
            cp1 = pltpu.async_copy(fw_hbm.at[idx1], g1, sem1)
            cp2 = pltpu.async_copy(fw_hbm.at[idx2], g2, sem2)
            cp0.wait()
            cp1.wait()
            cp2.wait()

            def _edge(e, _):
                r = [
                    jnp.maximum(
                        jnp.maximum(g0[e, pl.ds(16 * k, 16)],
                                    g1[e, pl.ds(16 * k, 16)]),
                        g2[e, pl.ds(16 * k, 16)])
                    for k in range(_D // 16)
                ]
                sv = r[0] * av[0] + r[1] * av[1] + r[2] * av[2] + r[3] * av[3]
                a = jnp.sum(sv)
                va = jnp.full((16,), a)
                p = jnp.exp(jnp.where(va > 0, va, va * 0.01))
                for k in range(_D // 16):
                    row_buf[e, pl.ds(16 * k, 16)] = r[k] * p
                row_buf[e, pl.ds(_D, 16)] = jnp.where(lane == 0, p, zvec)
                return 0

            lax.fori_loop(0, _CH, _edge, 0)
            pltpu.sync_copy(row_buf, acc.at[dst_buf], add=True)
            return 0

        lax.fori_loop(0, _NCHUNK, _chunk, 0)
        plsc.subcore_barrier()

        for k in range(_NZ // _ZB):
            pltpu.sync_copy(acc.at[pl.ds(s * _NZ + k * _ZB, _ZB)],
                            out_hbm.at[h, pl.ds(s * _NZ + k * _ZB, _ZB)])
        if q + 1 < _NPH:
            plsc.subcore_barrier()


_sc_kernel = functools.partial(
    pl.kernel,
    _sc_body,
    out_type=jax.ShapeDtypeStruct((_H, _N, _ROWW), jnp.float32),
    mesh=plsc.VectorSubcoreMesh(core_axis_name="c", subcore_axis_name="s",
                                num_cores=_NC, num_subcores=_NS),
    scratch_types=[
        pltpu.VMEM((_CH,), jnp.int32),
        pltpu.VMEM((_CH,), jnp.int32),
        pltpu.VMEM((_CH,), jnp.int32),
        pltpu.VMEM((_CH, _D), jnp.float32),
        pltpu.VMEM((_CH, _D), jnp.float32),
        pltpu.VMEM((_CH, _D), jnp.float32),
        pltpu.VMEM((_CH, _ROWW), jnp.float32),
        pltpu.VMEM((_CH,), jnp.int32),
        pltpu.VMEM((2 * _D,), jnp.float32),
        pltpu.VMEM((_ZB, _ROWW), jnp.float32),
        pltpu.VMEM_SHARED((_N, _ROWW), jnp.float32),
        pltpu.SemaphoreType.DMA,
        pltpu.SemaphoreType.DMA,
        pltpu.SemaphoreType.DMA,
    ],
    compiler_params=pltpu.CompilerParams(use_tc_tiling_on_sc=False,
                                         needs_layout_passes=False),
)


def kernel(features, edge_index, type_mask, edge_metapath_indices,
           W_rnn, b_rnn, attn):
    del type_mask
    n, d = features.shape
    e, l = edge_metapath_indices.shape
    h = attn.shape[1]
    assert (n, e, l, h, d) == (_N, _E, _L, _H, _D)

    fw = _fw_linear(features, W_rnn, b_rnn)
    fw4 = fw.reshape(_N * _H, _D)
    emi0 = edge_metapath_indices[:, 0]
    emi1 = edge_metapath_indices[:, 1]
    emi2 = edge_metapath_indices[:, 2]
    dst = edge_index[1]
    attn_flat = attn.reshape(_HD)

    acc = _sc_kernel()(fw4, emi0, emi1, emi2, dst, attn_flat)
    out = _finalize(acc)
    return out.reshape(_N, _H, _D)


# speedup vs baseline: 25.1865x; 1.8418x over previous
import functools

import jax
import jax.numpy as jnp
from jax import lax
from jax.experimental import pallas as pl
from jax.experimental.pallas import tpu as pltpu
from jax.experimental.pallas import tpu_sc as plsc

_N = 10000
_E = 160000
_L = 3
_H = 4
_D = 64
_HD = _H * _D
_NC = 2
_NS = 16
_NPH = _H // _NC
_EPS = _E // _NS
_CH = 40
_BLK = 2000
_NBLK = _EPS // _BLK
_CPB = _BLK // _CH
_ROWW = 80
_NZ = _N // _NS
_ZB = 25


def _fw_body(f_ref, w_ref, b_ref, o_ref):
    o_ref[...] = (
        jnp.dot(f_ref[...], w_ref[...], preferred_element_type=jnp.float32)
        + b_ref[...]
    )


def _fw_linear(features, w, b):
    n = features.shape[0]
    bn = 1000
    return pl.pallas_call(
        _fw_body,
        grid=(n // bn,),
        in_specs=[
            pl.BlockSpec((bn, _D), lambda i: (i, 0)),
            pl.BlockSpec((_D, _HD), lambda i: (0, 0)),
            pl.BlockSpec((_HD,), lambda i: (0,)),
        ],
        out_specs=pl.BlockSpec((bn, _HD), lambda i: (i, 0)),
        out_shape=jax.ShapeDtypeStruct((n, _HD), jnp.float32),
    )(features, w, b)


def _fin_body(acc_ref, o_ref):
    for h in range(_H):
        num = acc_ref[h, :, :_D]
        den = acc_ref[h, :, _D:_D + 1] + 1e-16
        o_ref[:, _D * h:_D * (h + 1)] = num / den


def _finalize(acc):
    bn = 1000
    return pl.pallas_call(
        _fin_body,
        grid=(_N // bn,),
        in_specs=[pl.BlockSpec((_H, bn, _ROWW), lambda i: (0, i, 0))],
        out_specs=pl.BlockSpec((bn, _HD), lambda i: (i, 0)),
        out_shape=jax.ShapeDtypeStruct((_N, _HD), jnp.float32),
    )(acc)


def _sc_body(fw_hbm, emi0_hbm, emi1_hbm, emi2_hbm, dst_hbm, attn_hbm,
             out_hbm,
             ia0, ia1, ia2, dst_all,
             g0a, g1a, g2a, g0b, g1b, g2b, row_a, row_b, attn_v,
             zero_buf, acc,
             sga, sgb, ssa, ssb, sidx):
    c = lax.axis_index("c")
    s = lax.axis_index("s")

    pltpu.sync_copy(attn_hbm.at[pl.ds(c * 2 * _D, 2 * _D)], attn_v)
    lane = lax.iota(jnp.int32, 16)
    zvec = jnp.zeros((16,), jnp.float32)

    def _zrow(r, _):
        for k in range(_ROWW // 16):
            zero_buf[r, pl.ds(16 * k, 16)] = zvec
        return 0
    lax.fori_loop(0, _ZB, _zrow, 0)

    def _gather_start(ib, t, half, g, sem):
        pltpu.make_async_copy(
            fw_hbm.at[ib.at[t, pl.ds(half * _CH, _CH)]], g, sem).start()

    def _g3_start(t, half, g0, g1, g2, sem):
        _gather_start(ia0, t, half, g0, sem)
        _gather_start(ia1, t, half, g1, sem)
        _gather_start(ia2, t, half, g2, sem)

    def _g3_wait(g0, g1, g2, sem):
        pltpu.make_async_copy(fw_hbm.at[ia0.at[0, pl.ds(0, _CH)]],
                              g0, sem).wait()
        pltpu.make_async_copy(fw_hbm.at[ia1.at[0, pl.ds(0, _CH)]],
                              g1, sem).wait()
        pltpu.make_async_copy(fw_hbm.at[ia2.at[0, pl.ds(0, _CH)]],
                              g2, sem).wait()

    def _scat_start(row, i, sem):
        pltpu.async_copy(row, acc.at[dst_all.at[i]], sem, add=True)

    def _scat_wait(row, sem):
        pltpu.make_async_copy(row, acc.at[dst_all.at[0]], sem).wait()

    for q in range(_NPH):
        h = 2 * c + q
        av = [attn_v[pl.ds(_D * q + 16 * k, 16)] for k in range(_D // 16)]

        for k in range(_NZ // _ZB):
            pltpu.sync_copy(zero_buf, acc.at[pl.ds(s * _NZ + k * _ZB, _ZB)])
        plsc.subcore_barrier()

        def _compute(g0, g1, g2, row):
            def _edge2(e2, _):
                for e in (2 * e2, 2 * e2 + 1):
                    r = [
                        jnp.maximum(
                            jnp.maximum(g0[e, pl.ds(16 * k, 16)],
                                        g1[e, pl.ds(16 * k, 16)]),
                            g2[e, pl.ds(16 * k, 16)])
                        for k in range(_D // 16)
                    ]
                    sv = (r[0] * av[0] + r[1] * av[1]
                          + r[2] * av[2] + r[3] * av[3])
                    a = jnp.sum(sv)
                    va = jnp.full((16,), a)
                    p = jnp.exp(jnp.where(va > 0, va, va * 0.01))
                    for k in range(_D // 16):
                        row[e, pl.ds(16 * k, 16)] = r[k] * p
                    row[e, pl.ds(_D, 16)] = jnp.where(lane == 0, p, zvec)
                return 0

            lax.fori_loop(0, _CH // 2, _edge2, 0)

        for blk in range(_NBLK):
            row0 = (s * _NBLK + blk) * (_BLK // 80)
            for src, ib in ((emi0_hbm, ia0), (emi1_hbm, ia1),
                            (emi2_hbm, ia2)):
                pltpu.make_async_copy(
                    src.at[pl.ds(row0, _BLK // 80)], ib, sidx).start()
            drow0 = (s * _NBLK + blk) * _CPB
            pltpu.make_async_copy(
                dst_hbm.at[pl.ds(drow0, _CPB)], dst_all, sidx).start()
            for src, ib in ((emi0_hbm, ia0), (emi1_hbm, ia1),
                            (emi2_hbm, ia2)):
                pltpu.make_async_copy(
                    src.at[pl.ds(row0, _BLK // 80)], ib, sidx).wait()
            pltpu.make_async_copy(
                dst_hbm.at[pl.ds(drow0, _CPB)], dst_all, sidx).wait()

            def _xform(r, _):
                for ib in (ia0, ia1, ia2):
                    for k in range(80 // 16):
                        ib[r, pl.ds(16 * k, 16)] = (
                            ib[r, pl.ds(16 * k, 16)] * 4 + h)
                return 0
            lax.fori_loop(0, _BLK // 80, _xform, 0)

            _g3_start(0, 0, g0a, g1a, g2a, sga)
            _g3_start(0, 1, g0b, g1b, g2b, sgb)

            def _pair(t, _):
                i0 = 2 * t
                _g3_wait(g0a, g1a, g2a, sga)

                @pl.when(t > 0)
                def _():
                    _scat_wait(row_a, ssa)
                _compute(g0a, g1a, g2a, row_a)
                _scat_start(row_a, i0, ssa)

                @pl.when(i0 + 2 < _CPB)
                def _():
                    _g3_start(t + 1, 0, g0a, g1a, g2a, sga)

                _g3_wait(g0b, g1b, g2b, sgb)

                @pl.when(t > 0)
                def _():
                    _scat_wait(row_b, ssb)
                _compute(g0b, g1b, g2b, row_b)
                _scat_start(row_b, i0 + 1, ssb)

                @pl.when(i0 + 3 < _CPB)
                def _():
                    _g3_start(t + 1, 1, g0b, g1b, g2b, sgb)
                return 0

            lax.fori_loop(0, _CPB // 2, _pair, 0)
            _scat_wait(row_a, ssa)
            _scat_wait(row_b, ssb)

        plsc.subcore_barrier()

        for k in range(_NZ // _ZB):
            pltpu.sync_copy(acc.at[pl.ds(s * _NZ + k * _ZB, _ZB)],
                            out_hbm.at[h, pl.ds(s * _NZ + k * _ZB, _ZB)])
        if q + 1 < _NPH:
            plsc.subcore_barrier()


_sc_kernel = functools.partial(
    pl.kernel,
    _sc_body,
    out_type=jax.ShapeDtypeStruct((_H, _N, _ROWW), jnp.float32),
    mesh=plsc.VectorSubcoreMesh(core_axis_name="c", subcore_axis_name="s",
                                num_cores=_NC, num_subcores=_NS),
    scratch_types=[
        pltpu.VMEM((_BLK // 80, 80), jnp.int32),
        pltpu.VMEM((_BLK // 80, 80), jnp.int32),
        pltpu.VMEM((_BLK // 80, 80), jnp.int32),
        pltpu.VMEM((_CPB, _CH), jnp.int32),
        pltpu.VMEM((_CH, _D), jnp.float32),
        pltpu.VMEM((_CH, _D), jnp.float32),
        pltpu.VMEM((_CH, _D), jnp.float32),
        pltpu.VMEM((_CH, _D), jnp.float32),
        pltpu.VMEM((_CH, _D), jnp.float32),
        pltpu.VMEM((_CH, _D), jnp.float32),
        pltpu.VMEM((_CH, _ROWW), jnp.float32),
        pltpu.VMEM((_CH, _ROWW), jnp.float32),
        pltpu.VMEM((2 * _D,), jnp.float32),
        pltpu.VMEM((_ZB, _ROWW), jnp.float32),
        pltpu.VMEM_SHARED((_N, _ROWW), jnp.float32),
        pltpu.SemaphoreType.DMA,
        pltpu.SemaphoreType.DMA,
        pltpu.SemaphoreType.DMA,
        pltpu.SemaphoreType.DMA,
        pltpu.SemaphoreType.DMA,
    ],
    compiler_params=pltpu.CompilerParams(use_tc_tiling_on_sc=False,
                                         needs_layout_passes=False),
)


def kernel(features, edge_index, type_mask, edge_metapath_indices,
           W_rnn, b_rnn, attn):
    del type_mask
    n, d = features.shape
    e, l = edge_metapath_indices.shape
    h = attn.shape[1]
    assert (n, e, l, h, d) == (_N, _E, _L, _H, _D)

    fw = _fw_linear(features, W_rnn, b_rnn)
    fw4 = fw.reshape(_N * _H, _D)
    emi0 = edge_metapath_indices[:, 0].reshape(_E // 80, 80)
    emi1 = edge_metapath_indices[:, 1].reshape(_E // 80, 80)
    emi2 = edge_metapath_indices[:, 2].reshape(_E // 80, 80)
    dst = edge_index[1].reshape(_E // _CH, _CH)
    attn_flat = attn.reshape(_HD)

    acc = _sc_kernel()(fw4, emi0, emi1, emi2, dst, attn_flat)
    out = _finalize(acc)
    return out.reshape(_N, _H, _D)
